# Initial kernel scaffold; baseline (speedup 1.0000x reference)
import jax
import jax.numpy as jnp
from jax.experimental import pallas as pl


def kernel(a, b, sa, sba):
    raise NotImplementedError("write your pallas kernel here")


# speedup vs baseline: 59.7746x; 59.7746x over previous
import jax
import jax.numpy as jnp
from jax import lax
from jax.experimental import pallas as pl
from jax.experimental.pallas import tpu as pltpu
from jax.experimental.pallas import tpu_sc as plsc

K = 8192
B = 16384
L = 200

N = B * L
LANES = 16
NW = 32
PER_W = N // NW
CHUNK = 4096
NCHUNKS = PER_W // CHUNK


def _body(a_hbm, b_hbm, sa_hbm, sba_hbm, out_hbm,
          a_v, b_v, idx_v, val_v, sa_v, sem):
    cid = lax.axis_index("c")
    sid = lax.axis_index("s")
    wid = sid * 2 + cid
    base = wid * PER_W

    pltpu.sync_copy(sa_hbm, sa_v)

    def chunk(g, carry):
        off = base + g * CHUNK
        pltpu.sync_copy(a_hbm.at[pl.ds(off, CHUNK)], a_v)
        pltpu.sync_copy(b_hbm.at[pl.ds(off, CHUNK)], b_v)

        def mkidx(i, c2):
            sl = pl.ds(i * LANES, LANES)
            idx_v[sl] = a_v[sl] * K + b_v[sl]
            return c2

        lax.fori_loop(0, CHUNK // LANES, mkidx, 0)

        pltpu.async_copy(sba_hbm.at[idx_v], val_v, sem).wait()

        def addsa(i, c2):
            sl = pl.ds(i * LANES, LANES)
            val_v[sl] = val_v[sl] + plsc.load_gather(sa_v, [a_v[sl]])
            return c2

        lax.fori_loop(0, CHUNK // LANES, addsa, 0)

        pltpu.sync_copy(val_v, out_hbm.at[pl.ds(off, CHUNK)])
        return carry

    lax.fori_loop(0, NCHUNKS, chunk, 0)


@jax.jit
def kernel(a, b, sa, sba):
    af = a.reshape(N)
    bf = b.reshape(N)
    sba_flat = sba.reshape(K * K)

    mesh = plsc.VectorSubcoreMesh(core_axis_name="c", subcore_axis_name="s")
    out = pl.kernel(
        _body,
        out_type=jax.ShapeDtypeStruct((N,), jnp.float32),
        mesh=mesh,
        compiler_params=pltpu.CompilerParams(needs_layout_passes=False),
        scratch_types=[
            pltpu.VMEM((CHUNK,), jnp.int32),
            pltpu.VMEM((CHUNK,), jnp.int32),
            pltpu.VMEM((CHUNK,), jnp.int32),
            pltpu.VMEM((CHUNK,), jnp.float32),
            pltpu.VMEM((K,), jnp.float32),
            pltpu.SemaphoreType.DMA,
        ],
    )(af, bf, sa, sba_flat)
    return out.reshape(B, L)


# speedup vs baseline: 73.2618x; 1.2256x over previous
import jax
import jax.numpy as jnp
from jax import lax
from jax.experimental import pallas as pl
from jax.experimental.pallas import tpu as pltpu
from jax.experimental.pallas import tpu_sc as plsc

K = 8192
B = 16384
L = 200

N = B * L
LANES = 16
NW = 32
PER_W = N // NW
CHUNK = 6400
NCHUNKS = PER_W // CHUNK
NVEC = CHUNK // LANES


def _body(a_hbm, b_hbm, sa_hbm, sba_hbm, out_hbm,
          a0, b0, i0, v0, a1, b1, i1, v1, sa_v,
          si0, si1, sg0, sg1, so0, so1):
    cid = lax.axis_index("c")
    sid = lax.axis_index("s")
    wid = sid * 2 + cid
    base = wid * PER_W

    bufs = ((a0, b0, i0, v0), (a1, b1, i1, v1))
    in_sems = (si0, si1)
    g_sems = (sg0, sg1)
    o_sems = (so0, so1)

    pltpu.sync_copy(sa_hbm, sa_v)

    def start_in(g, p):
        off = base + g * CHUNK
        a_p, b_p, _, _ = bufs[p]
        pltpu.async_copy(a_hbm.at[pl.ds(off, CHUNK)], a_p, in_sems[p])
        pltpu.async_copy(b_hbm.at[pl.ds(off, CHUNK)], b_p, in_sems[p])

    def wait_in(p):
        a_p, b_p, _, _ = bufs[p]
        pltpu.make_async_copy(a_hbm.at[pl.ds(0, CHUNK)], a_p, in_sems[p]).wait()
        pltpu.make_async_copy(b_hbm.at[pl.ds(0, CHUNK)], b_p, in_sems[p]).wait()

    def loop1(p):
        a_p, b_p, i_p, _ = bufs[p]

        def f(i, c):
            sl = pl.ds(i * LANES, LANES)
            i_p[sl] = a_p[sl] * K + b_p[sl]
            return c

        lax.fori_loop(0, NVEC, f, 0)

    def start_gather(p):
        _, _, i_p, v_p = bufs[p]
        pltpu.async_copy(sba_hbm.at[i_p], v_p, g_sems[p])

    def wait_gather(p):
        _, _, i_p, v_p = bufs[p]
        pltpu.make_async_copy(sba_hbm.at[i_p], v_p, g_sems[p]).wait()

    def loop2(p):
        a_p, _, _, v_p = bufs[p]

        def f(i, c):
            sl = pl.ds(i * LANES, LANES)
            v_p[sl] = v_p[sl] + plsc.load_gather(sa_v, [a_p[sl]])
            return c

        lax.fori_loop(0, NVEC, f, 0)

    def start_out(g, p):
        off = base + g * CHUNK
        _, _, _, v_p = bufs[p]
        pltpu.async_copy(v_p, out_hbm.at[pl.ds(off, CHUNK)], o_sems[p])

    def wait_out(p):
        _, _, _, v_p = bufs[p]
        pltpu.make_async_copy(v_p, out_hbm.at[pl.ds(0, CHUNK)], o_sems[p]).wait()

    start_in(0, 0)
    for g in range(NCHUNKS):
        p = g & 1
        q = 1 - p
        wait_in(p)
        loop1(p)
        if g >= 2:
            wait_out(p)
        start_gather(p)
        if g >= 1:
            wait_gather(q)
            loop2(q)
            start_out(g - 1, q)
        if g + 1 < NCHUNKS:
            start_in(g + 1, q)
    p = (NCHUNKS - 1) & 1
    wait_gather(p)
    loop2(p)
    start_out(NCHUNKS - 1, p)
    wait_out(1 - p)
    wait_out(p)


@jax.jit
def kernel(a, b, sa, sba):
    af = a.reshape(N)
    bf = b.reshape(N)
    sba_flat = sba.reshape(K * K)

    mesh = plsc.VectorSubcoreMesh(core_axis_name="c", subcore_axis_name="s")
    out = pl.kernel(
        _body,
        out_type=jax.ShapeDtypeStruct((N,), jnp.float32),
        mesh=mesh,
        compiler_params=pltpu.CompilerParams(needs_layout_passes=False),
        scratch_types=[
            pltpu.VMEM((CHUNK,), jnp.int32),
            pltpu.VMEM((CHUNK,), jnp.int32),
            pltpu.VMEM((CHUNK,), jnp.int32),
            pltpu.VMEM((CHUNK,), jnp.float32),
            pltpu.VMEM((CHUNK,), jnp.int32),
            pltpu.VMEM((CHUNK,), jnp.int32),
            pltpu.VMEM((CHUNK,), jnp.int32),
            pltpu.VMEM((CHUNK,), jnp.float32),
            pltpu.VMEM((K,), jnp.float32),
            pltpu.SemaphoreType.DMA,
            pltpu.SemaphoreType.DMA,
            pltpu.SemaphoreType.DMA,
            pltpu.SemaphoreType.DMA,
            pltpu.SemaphoreType.DMA,
            pltpu.SemaphoreType.DMA,
        ],
    )(af, bf, sa, sba_flat)
    return out.reshape(B, L)
